# baseline (speedup 1.0000x reference)
import functools

import jax
import jax.numpy as jnp
from jax.experimental import pallas as pl
from jax.experimental.pallas import tpu as pltpu
from jax.experimental.pallas import tpu_sc as plsc

B = 4096
D = 512
R = 256
NBLK = B // R
PAD = 128

_K1 = (1832780943, 270669613)
_K2 = (64467757, 2916123636)
_TINY = float(jnp.finfo(jnp.float32).tiny)
_ROTS = ((13, 15, 26, 6), (17, 29, 16, 24))


def _gumbel_from_bits(bits):
    fb = jax.lax.shift_right_logical(bits, jnp.uint32(9)) | jnp.uint32(0x3F800000)
    f = jax.lax.bitcast_convert_type(fb, jnp.float32) - 1.0
    u = jnp.maximum(jnp.float32(_TINY), f * 1.0 + jnp.float32(_TINY))
    return -jnp.log(-jnp.log(u))


_CH = 8
_CW = 512


def _threefry_chunks(r0, bs1_ref, bs2_ref):
    def sched(key):
        k0 = jnp.uint32(key[0])
        k1 = jnp.uint32(key[1])
        return (k0, k1, k0 ^ k1 ^ jnp.uint32(0x1BD11BDA))

    ksa, ksb = sched(_K1), sched(_K2)
    col_c = jax.lax.broadcasted_iota(jnp.int32, (_CH, _CW), 1)
    row_c = jax.lax.broadcasted_iota(jnp.int32, (_CH, _CW), 0)
    ncw = B // _CW

    def body(jk, carry):
        j = jk // ncw
        k = jk % ncw
        n = ((r0 + j * _CH + row_c) * B + (k * _CW + col_c)).astype(jnp.uint32)
        xa0 = jnp.full((_CH, _CW), ksa[0], jnp.uint32)
        xb0 = jnp.full((_CH, _CW), ksb[0], jnp.uint32)
        xa1 = n + ksa[1]
        xb1 = n + ksb[1]
        for i in range(5):
            for r in _ROTS[i % 2]:
                xa0 = xa0 + xa1
                xb0 = xb0 + xb1
                xa1 = (jax.lax.shift_left(xa1, jnp.uint32(r))
                       | jax.lax.shift_right_logical(xa1, jnp.uint32(32 - r))) ^ xa0
                xb1 = (jax.lax.shift_left(xb1, jnp.uint32(r))
                       | jax.lax.shift_right_logical(xb1, jnp.uint32(32 - r))) ^ xb0
            xa0 = xa0 + ksa[(i + 1) % 3]
            xb0 = xb0 + ksb[(i + 1) % 3]
            xa1 = xa1 + ksa[(i + 2) % 3] + jnp.uint32(i + 1)
            xb1 = xb1 + ksb[(i + 2) % 3] + jnp.uint32(i + 1)
        bs1_ref[pl.ds(j * _CH, _CH), pl.ds(k * _CW, _CW)] = xa0 ^ xa1
        bs2_ref[pl.ds(j * _CH, _CH), pl.ds(k * _CW, _CW)] = xb0 ^ xb1
        return carry

    jax.lax.fori_loop(0, (R // _CH) * ncw, body, 0)


def _sample_project_body(li_ref, lt_ref, ai_ref, at_ref,
                         pwi_ref, pwt_ref,
                         idxt_ref, idxi_ref, pi_ref, pt_ref,
                         bs1_ref, bs2_ref):
    i = pl.program_id(0)
    r0 = i * R

    _threefry_chunks(r0, bs1_ref, bs2_ref)

    col = jax.lax.broadcasted_iota(jnp.int32, (R, B), 1)
    row = r0 + jax.lax.broadcasted_iota(jnp.int32, (R, B), 0)
    diag = col == row

    def draw(x, bits):
        g = _gumbel_from_bits(bits)
        m = jnp.max(x, axis=1, keepdims=True)
        u = jnp.exp(x - m)
        s = jnp.sum(u, axis=1, keepdims=True)
        w = u / s
        w = jnp.where(diag, 0.0, w)
        v = jnp.where(w > 0, jnp.log(w), -jnp.inf) + g
        vmax = jnp.max(v, axis=1, keepdims=True)
        cand = jnp.where(v == vmax, col, B)
        return jnp.min(cand, axis=1).astype(jnp.int32)

    idxt_ref[0, pl.ds(r0, R)] = draw(li_ref[...], bs1_ref[...])
    idxi_ref[0, pl.ds(r0, R)] = draw(lt_ref[...], bs2_ref[...])

    pi_ref[...] = jnp.dot(ai_ref[...], pwi_ref[...],
                          preferred_element_type=jnp.float32)
    pt_ref[...] = jnp.dot(at_ref[...], pwt_ref[...],
                          preferred_element_type=jnp.float32)


_NW = 32
_BPW = B // _NW


def _sc_gather_body(pt_hbm, pi_hbm, idxt_hbm, idxi_hbm,
                    outt_hbm, outi_hbm, idx_v, rows_v, sem):
    wid = jax.lax.axis_index("s") * 2 + jax.lax.axis_index("c")
    base = wid * _BPW
    pltpu.sync_copy(idxt_hbm.at[pl.ds(base, _BPW)], idx_v)
    pltpu.async_copy(pt_hbm.at[idx_v], rows_v, sem).wait()
    pltpu.sync_copy(rows_v, outt_hbm.at[pl.ds(base, _BPW)])
    pltpu.sync_copy(idxi_hbm.at[pl.ds(base, _BPW)], idx_v)
    pltpu.async_copy(pi_hbm.at[idx_v], rows_v, sem).wait()
    pltpu.sync_copy(rows_v, outi_hbm.at[pl.ds(base, _BPW)])


def _finalize_body(pi_ref, pt_ref, gt_ref, gi_ref, pb_ref,
                   lg0_ref, lg1_ref, lg2_ref, loss_ref):
    i = pl.program_id(0)

    gath_t = gt_ref[...]
    gath_i = gi_ref[...]
    pi_blk = pi_ref[...]
    pt_blk = pt_ref[...]
    pb = pb_ref[...]

    lg0 = pi_blk + pt_blk + pb
    lg1 = pi_blk + gath_t + pb
    lg2 = gath_i + pt_blk + pb
    lg0_ref[...] = lg0
    lg1_ref[...] = lg1
    lg2_ref[...] = lg2

    def logp(lg, want_pos):
        a = lg[:, 0:1]
        b = lg[:, 1:2]
        mx = jnp.maximum(a, b)
        lse = jnp.log(jnp.exp(a - mx) + jnp.exp(b - mx))
        sel = b if want_pos else a
        return (sel - mx) - lse

    partial = (jnp.sum(logp(lg0, True)) + jnp.sum(logp(lg1, False))
               + jnp.sum(logp(lg2, False)))

    @pl.when(i == 0)
    def _():
        loss_ref[...] = jnp.zeros_like(loss_ref)

    loss_ref[...] += jnp.full((1, 1), partial, jnp.float32)

    @pl.when(i == NBLK - 1)
    def _():
        loss_ref[...] = loss_ref[...] * (-1.0 / (3.0 * B))


@functools.partial(jax.jit, static_argnames=())
def kernel(all_image_features, all_text_features, logits_per_image,
           logits_per_text, proj_w, proj_b):
    pw_img = jnp.zeros((D, PAD), jnp.float32).at[:, :2].set(proj_w[:D])
    pw_txt = jnp.zeros((D, PAD), jnp.float32).at[:, :2].set(proj_w[D:])
    pb_pad = jnp.zeros((1, PAD), jnp.float32).at[0, :2].set(proj_b)

    row_spec = pl.BlockSpec((R, B), lambda i: (i, 0))
    feat_spec = pl.BlockSpec((R, D), lambda i: (i, 0))
    full_w = pl.BlockSpec((D, PAD), lambda i: (0, 0))
    idx_spec = pl.BlockSpec((1, B), lambda i: (0, 0))
    proj_out = pl.BlockSpec((R, PAD), lambda i: (i, 0))

    idxt, idxi, pi, pt = pl.pallas_call(
        _sample_project_body,
        grid=(NBLK,),
        in_specs=[row_spec, row_spec,
                  feat_spec, feat_spec, full_w, full_w],
        out_specs=[idx_spec, idx_spec, proj_out, proj_out],
        out_shape=[
            jax.ShapeDtypeStruct((1, B), jnp.int32),
            jax.ShapeDtypeStruct((1, B), jnp.int32),
            jax.ShapeDtypeStruct((B, PAD), jnp.float32),
            jax.ShapeDtypeStruct((B, PAD), jnp.float32),
        ],
        scratch_shapes=[pltpu.VMEM((R, B), jnp.uint32),
                        pltpu.VMEM((R, B), jnp.uint32)],
    )(logits_per_image, logits_per_text,
      all_image_features, all_text_features, pw_img, pw_txt)

    sc_gather = functools.partial(
        pl.kernel,
        mesh=plsc.VectorSubcoreMesh(core_axis_name="c", subcore_axis_name="s"),
        out_type=[
            jax.ShapeDtypeStruct((B, PAD), jnp.float32),
            jax.ShapeDtypeStruct((B, PAD), jnp.float32),
        ],
        scratch_types=[
            pltpu.VMEM((_BPW,), jnp.int32),
            pltpu.VMEM((_BPW, PAD), jnp.float32),
            pltpu.SemaphoreType.DMA,
        ],
    )(_sc_gather_body)
    gath_t, gath_i = sc_gather(pt, pi, idxt.reshape(B), idxi.reshape(B))

    pb_spec = pl.BlockSpec((1, PAD), lambda i: (0, 0))
    lg_spec = pl.BlockSpec((R, PAD), lambda i: (i, 0))
    loss_spec = pl.BlockSpec((1, 1), lambda i: (0, 0))

    lg0, lg1, lg2, loss = pl.pallas_call(
        _finalize_body,
        grid=(NBLK,),
        in_specs=[lg_spec, lg_spec, lg_spec, lg_spec, pb_spec],
        out_specs=[lg_spec, lg_spec, lg_spec, loss_spec],
        out_shape=[
            jax.ShapeDtypeStruct((B, PAD), jnp.float32),
            jax.ShapeDtypeStruct((B, PAD), jnp.float32),
            jax.ShapeDtypeStruct((B, PAD), jnp.float32),
            jax.ShapeDtypeStruct((1, 1), jnp.float32),
        ],
    )(pi, pt, gath_t, gath_i, pb_pad)

    logits = jnp.concatenate([lg0[:, :2], lg1[:, :2], lg2[:, :2]], axis=0)
    itm_labels = jnp.concatenate([
        jnp.ones((B,), dtype=jnp.int32),
        jnp.zeros((B,), dtype=jnp.int32),
        jnp.zeros((B,), dtype=jnp.int32),
    ])
    return loss[0, 0], logits, itm_labels


# speedup vs baseline: 1.1546x; 1.1546x over previous
import functools

import jax
import jax.numpy as jnp
from jax.experimental import pallas as pl
from jax.experimental.pallas import tpu as pltpu
from jax.experimental.pallas import tpu_sc as plsc

B = 4096
D = 512
R = 256
NBLK = B // R
PAD = 128

_K1 = (1832780943, 270669613)
_K2 = (64467757, 2916123636)
_TINY = float(jnp.finfo(jnp.float32).tiny)
_ROTS = ((13, 15, 26, 6), (17, 29, 16, 24))


def _gumbel_from_bits(bits):
    fb = jax.lax.shift_right_logical(bits, jnp.uint32(9)) | jnp.uint32(0x3F800000)
    f = jax.lax.bitcast_convert_type(fb, jnp.float32) - 1.0
    u = jnp.maximum(jnp.float32(_TINY), f * 1.0 + jnp.float32(_TINY))
    return -jnp.log(-jnp.log(u))


_CH = 16
_CW = 4096


def _threefry_chunks(r0, bs1_ref, bs2_ref):
    def sched(key):
        k0 = jnp.uint32(key[0])
        k1 = jnp.uint32(key[1])
        return (k0, k1, k0 ^ k1 ^ jnp.uint32(0x1BD11BDA))

    ksa, ksb = sched(_K1), sched(_K2)
    col_c = jax.lax.broadcasted_iota(jnp.int32, (_CH, _CW), 1)
    row_c = jax.lax.broadcasted_iota(jnp.int32, (_CH, _CW), 0)
    ncw = B // _CW

    def body(jk, carry):
        j = jk // ncw
        k = jk % ncw
        n = ((r0 + j * _CH + row_c) * B + (k * _CW + col_c)).astype(jnp.uint32)
        xa0 = jnp.full((_CH, _CW), ksa[0], jnp.uint32)
        xb0 = jnp.full((_CH, _CW), ksb[0], jnp.uint32)
        xa1 = n + ksa[1]
        xb1 = n + ksb[1]
        for i in range(5):
            for r in _ROTS[i % 2]:
                xa0 = xa0 + xa1
                xb0 = xb0 + xb1
                xa1 = (jax.lax.shift_left(xa1, jnp.uint32(r))
                       | jax.lax.shift_right_logical(xa1, jnp.uint32(32 - r))) ^ xa0
                xb1 = (jax.lax.shift_left(xb1, jnp.uint32(r))
                       | jax.lax.shift_right_logical(xb1, jnp.uint32(32 - r))) ^ xb0
            xa0 = xa0 + ksa[(i + 1) % 3]
            xb0 = xb0 + ksb[(i + 1) % 3]
            xa1 = xa1 + ksa[(i + 2) % 3] + jnp.uint32(i + 1)
            xb1 = xb1 + ksb[(i + 2) % 3] + jnp.uint32(i + 1)
        bs1_ref[pl.ds(j * _CH, _CH), pl.ds(k * _CW, _CW)] = xa0 ^ xa1
        bs2_ref[pl.ds(j * _CH, _CH), pl.ds(k * _CW, _CW)] = xb0 ^ xb1
        return carry

    jax.lax.fori_loop(0, (R // _CH) * ncw, body, 0)


def _sample_project_body(li_ref, lt_ref, ai_ref, at_ref,
                         pwi_ref, pwt_ref,
                         idxt_ref, idxi_ref, pi_ref, pt_ref,
                         bs1_ref, bs2_ref):
    i = pl.program_id(0)
    r0 = i * R

    _threefry_chunks(r0, bs1_ref, bs2_ref)

    col = jax.lax.broadcasted_iota(jnp.int32, (R, B), 1)
    row = r0 + jax.lax.broadcasted_iota(jnp.int32, (R, B), 0)
    diag = col == row

    def draw(x, bits):
        g = _gumbel_from_bits(bits)
        m = jnp.max(x, axis=1, keepdims=True)
        u = jnp.exp(x - m)
        s = jnp.sum(u, axis=1, keepdims=True)
        w = u / s
        w = jnp.where(diag, 0.0, w)
        v = jnp.where(w > 0, jnp.log(w), -jnp.inf) + g
        vmax = jnp.max(v, axis=1, keepdims=True)
        cand = jnp.where(v == vmax, col, B)
        return jnp.min(cand, axis=1).astype(jnp.int32)

    idxt_ref[0, pl.ds(r0, R)] = draw(li_ref[...], bs1_ref[...])
    idxi_ref[0, pl.ds(r0, R)] = draw(lt_ref[...], bs2_ref[...])

    pi_ref[...] = jnp.dot(ai_ref[...], pwi_ref[...],
                          preferred_element_type=jnp.float32)
    pt_ref[...] = jnp.dot(at_ref[...], pwt_ref[...],
                          preferred_element_type=jnp.float32)


_NW = 32
_BPW = B // _NW


def _sc_gather_body(pt_hbm, pi_hbm, idxt_hbm, idxi_hbm,
                    outt_hbm, outi_hbm, idx_v, rows_v, sem):
    wid = jax.lax.axis_index("s") * 2 + jax.lax.axis_index("c")
    base = wid * _BPW
    pltpu.sync_copy(idxt_hbm.at[pl.ds(base, _BPW)], idx_v)
    pltpu.async_copy(pt_hbm.at[idx_v], rows_v, sem).wait()
    pltpu.sync_copy(rows_v, outt_hbm.at[pl.ds(base, _BPW)])
    pltpu.sync_copy(idxi_hbm.at[pl.ds(base, _BPW)], idx_v)
    pltpu.async_copy(pi_hbm.at[idx_v], rows_v, sem).wait()
    pltpu.sync_copy(rows_v, outi_hbm.at[pl.ds(base, _BPW)])


def _finalize_body(pi_ref, pt_ref, gt_ref, gi_ref, pb_ref,
                   lg0_ref, lg1_ref, lg2_ref, loss_ref):
    i = pl.program_id(0)

    gath_t = gt_ref[...]
    gath_i = gi_ref[...]
    pi_blk = pi_ref[...]
    pt_blk = pt_ref[...]
    pb = pb_ref[...]

    lg0 = pi_blk + pt_blk + pb
    lg1 = pi_blk + gath_t + pb
    lg2 = gath_i + pt_blk + pb
    lg0_ref[...] = lg0
    lg1_ref[...] = lg1
    lg2_ref[...] = lg2

    def logp(lg, want_pos):
        a = lg[:, 0:1]
        b = lg[:, 1:2]
        mx = jnp.maximum(a, b)
        lse = jnp.log(jnp.exp(a - mx) + jnp.exp(b - mx))
        sel = b if want_pos else a
        return (sel - mx) - lse

    partial = (jnp.sum(logp(lg0, True)) + jnp.sum(logp(lg1, False))
               + jnp.sum(logp(lg2, False)))

    @pl.when(i == 0)
    def _():
        loss_ref[...] = jnp.zeros_like(loss_ref)

    loss_ref[...] += jnp.full((1, 1), partial, jnp.float32)

    @pl.when(i == NBLK - 1)
    def _():
        loss_ref[...] = loss_ref[...] * (-1.0 / (3.0 * B))


@functools.partial(jax.jit, static_argnames=())
def kernel(all_image_features, all_text_features, logits_per_image,
           logits_per_text, proj_w, proj_b):
    pw_img = jnp.zeros((D, PAD), jnp.float32).at[:, :2].set(proj_w[:D])
    pw_txt = jnp.zeros((D, PAD), jnp.float32).at[:, :2].set(proj_w[D:])
    pb_pad = jnp.zeros((1, PAD), jnp.float32).at[0, :2].set(proj_b)

    row_spec = pl.BlockSpec((R, B), lambda i: (i, 0))
    feat_spec = pl.BlockSpec((R, D), lambda i: (i, 0))
    full_w = pl.BlockSpec((D, PAD), lambda i: (0, 0))
    idx_spec = pl.BlockSpec((1, B), lambda i: (0, 0))
    proj_out = pl.BlockSpec((R, PAD), lambda i: (i, 0))

    idxt, idxi, pi, pt = pl.pallas_call(
        _sample_project_body,
        grid=(NBLK,),
        in_specs=[row_spec, row_spec,
                  feat_spec, feat_spec, full_w, full_w],
        out_specs=[idx_spec, idx_spec, proj_out, proj_out],
        out_shape=[
            jax.ShapeDtypeStruct((1, B), jnp.int32),
            jax.ShapeDtypeStruct((1, B), jnp.int32),
            jax.ShapeDtypeStruct((B, PAD), jnp.float32),
            jax.ShapeDtypeStruct((B, PAD), jnp.float32),
        ],
        scratch_shapes=[pltpu.VMEM((R, B), jnp.uint32),
                        pltpu.VMEM((R, B), jnp.uint32)],
    )(logits_per_image, logits_per_text,
      all_image_features, all_text_features, pw_img, pw_txt)

    sc_gather = functools.partial(
        pl.kernel,
        mesh=plsc.VectorSubcoreMesh(core_axis_name="c", subcore_axis_name="s"),
        out_type=[
            jax.ShapeDtypeStruct((B, PAD), jnp.float32),
            jax.ShapeDtypeStruct((B, PAD), jnp.float32),
        ],
        scratch_types=[
            pltpu.VMEM((_BPW,), jnp.int32),
            pltpu.VMEM((_BPW, PAD), jnp.float32),
            pltpu.SemaphoreType.DMA,
        ],
    )(_sc_gather_body)
    gath_t, gath_i = sc_gather(pt, pi, idxt.reshape(B), idxi.reshape(B))

    pb_spec = pl.BlockSpec((1, PAD), lambda i: (0, 0))
    lg_spec = pl.BlockSpec((R, PAD), lambda i: (i, 0))
    loss_spec = pl.BlockSpec((1, 1), lambda i: (0, 0))

    lg0, lg1, lg2, loss = pl.pallas_call(
        _finalize_body,
        grid=(NBLK,),
        in_specs=[lg_spec, lg_spec, lg_spec, lg_spec, pb_spec],
        out_specs=[lg_spec, lg_spec, lg_spec, loss_spec],
        out_shape=[
            jax.ShapeDtypeStruct((B, PAD), jnp.float32),
            jax.ShapeDtypeStruct((B, PAD), jnp.float32),
            jax.ShapeDtypeStruct((B, PAD), jnp.float32),
            jax.ShapeDtypeStruct((1, 1), jnp.float32),
        ],
    )(pi, pt, gath_t, gath_i, pb_pad)

    logits = jnp.concatenate([lg0[:, :2], lg1[:, :2], lg2[:, :2]], axis=0)
    itm_labels = jnp.concatenate([
        jnp.ones((B,), dtype=jnp.int32),
        jnp.zeros((B,), dtype=jnp.int32),
        jnp.zeros((B,), dtype=jnp.int32),
    ])
    return loss[0, 0], logits, itm_labels


# speedup vs baseline: 1.1624x; 1.0068x over previous
import functools

import jax
import jax.numpy as jnp
from jax.experimental import pallas as pl
from jax.experimental.pallas import tpu as pltpu
from jax.experimental.pallas import tpu_sc as plsc

B = 4096
D = 512
R = 256
NBLK = B // R
PAD = 128

_K1 = (1832780943, 270669613)
_K2 = (64467757, 2916123636)
_TINY = float(jnp.finfo(jnp.float32).tiny)
_ROTS = ((13, 15, 26, 6), (17, 29, 16, 24))


def _gumbel_from_bits(bits):
    fb = jax.lax.shift_right_logical(bits, jnp.uint32(9)) | jnp.uint32(0x3F800000)
    f = jax.lax.bitcast_convert_type(fb, jnp.float32) - 1.0
    u = jnp.maximum(jnp.float32(_TINY), f * 1.0 + jnp.float32(_TINY))
    return -jnp.log(-jnp.log(u))


_CH = 32
_CW = 4096


def _threefry_chunks(r0, bs1_ref, bs2_ref):
    def sched(key):
        k0 = jnp.uint32(key[0])
        k1 = jnp.uint32(key[1])
        return (k0, k1, k0 ^ k1 ^ jnp.uint32(0x1BD11BDA))

    ksa, ksb = sched(_K1), sched(_K2)
    col_c = jax.lax.broadcasted_iota(jnp.int32, (_CH, _CW), 1)
    row_c = jax.lax.broadcasted_iota(jnp.int32, (_CH, _CW), 0)
    ncw = B // _CW

    def body(jk, carry):
        j = jk // ncw
        k = jk % ncw
        n = ((r0 + j * _CH + row_c) * B + (k * _CW + col_c)).astype(jnp.uint32)
        xa0 = jnp.full((_CH, _CW), ksa[0], jnp.uint32)
        xb0 = jnp.full((_CH, _CW), ksb[0], jnp.uint32)
        xa1 = n + ksa[1]
        xb1 = n + ksb[1]
        for i in range(5):
            for r in _ROTS[i % 2]:
                xa0 = xa0 + xa1
                xb0 = xb0 + xb1
                xa1 = (jax.lax.shift_left(xa1, jnp.uint32(r))
                       | jax.lax.shift_right_logical(xa1, jnp.uint32(32 - r))) ^ xa0
                xb1 = (jax.lax.shift_left(xb1, jnp.uint32(r))
                       | jax.lax.shift_right_logical(xb1, jnp.uint32(32 - r))) ^ xb0
            xa0 = xa0 + ksa[(i + 1) % 3]
            xb0 = xb0 + ksb[(i + 1) % 3]
            xa1 = xa1 + ksa[(i + 2) % 3] + jnp.uint32(i + 1)
            xb1 = xb1 + ksb[(i + 2) % 3] + jnp.uint32(i + 1)
        bs1_ref[pl.ds(j * _CH, _CH), pl.ds(k * _CW, _CW)] = xa0 ^ xa1
        bs2_ref[pl.ds(j * _CH, _CH), pl.ds(k * _CW, _CW)] = xb0 ^ xb1
        return carry

    jax.lax.fori_loop(0, (R // _CH) * ncw, body, 0)


def _sample_project_body(li_ref, lt_ref, ai_ref, at_ref,
                         pwi_ref, pwt_ref,
                         idxt_ref, idxi_ref, pi_ref, pt_ref,
                         bs1_ref, bs2_ref):
    i = pl.program_id(0)
    r0 = i * R

    _threefry_chunks(r0, bs1_ref, bs2_ref)

    col = jax.lax.broadcasted_iota(jnp.int32, (R, B), 1)
    row = r0 + jax.lax.broadcasted_iota(jnp.int32, (R, B), 0)
    diag = col == row

    def draw(x, bits):
        g = _gumbel_from_bits(bits)
        m = jnp.max(x, axis=1, keepdims=True)
        u = jnp.exp(x - m)
        s = jnp.sum(u, axis=1, keepdims=True)
        w = u / s
        w = jnp.where(diag, 0.0, w)
        v = jnp.where(w > 0, jnp.log(w), -jnp.inf) + g
        vmax = jnp.max(v, axis=1, keepdims=True)
        cand = jnp.where(v == vmax, col, B)
        return jnp.min(cand, axis=1).astype(jnp.int32)

    idxt_ref[0, pl.ds(r0, R)] = draw(li_ref[...], bs1_ref[...])
    idxi_ref[0, pl.ds(r0, R)] = draw(lt_ref[...], bs2_ref[...])

    pi_ref[...] = jnp.dot(ai_ref[...], pwi_ref[...],
                          preferred_element_type=jnp.float32)
    pt_ref[...] = jnp.dot(at_ref[...], pwt_ref[...],
                          preferred_element_type=jnp.float32)


_NW = 32
_BPW = B // _NW


def _sc_gather_body(pt_hbm, pi_hbm, idxt_hbm, idxi_hbm,
                    outt_hbm, outi_hbm, idx_v, rows_v, sem):
    wid = jax.lax.axis_index("s") * 2 + jax.lax.axis_index("c")
    base = wid * _BPW
    pltpu.sync_copy(idxt_hbm.at[pl.ds(base, _BPW)], idx_v)
    pltpu.async_copy(pt_hbm.at[idx_v], rows_v, sem).wait()
    pltpu.sync_copy(rows_v, outt_hbm.at[pl.ds(base, _BPW)])
    pltpu.sync_copy(idxi_hbm.at[pl.ds(base, _BPW)], idx_v)
    pltpu.async_copy(pi_hbm.at[idx_v], rows_v, sem).wait()
    pltpu.sync_copy(rows_v, outi_hbm.at[pl.ds(base, _BPW)])


def _finalize_body(pi_ref, pt_ref, gt_ref, gi_ref, pb_ref,
                   lg0_ref, lg1_ref, lg2_ref, loss_ref):
    i = pl.program_id(0)

    gath_t = gt_ref[...]
    gath_i = gi_ref[...]
    pi_blk = pi_ref[...]
    pt_blk = pt_ref[...]
    pb = pb_ref[...]

    lg0 = pi_blk + pt_blk + pb
    lg1 = pi_blk + gath_t + pb
    lg2 = gath_i + pt_blk + pb
    lg0_ref[...] = lg0
    lg1_ref[...] = lg1
    lg2_ref[...] = lg2

    def logp(lg, want_pos):
        a = lg[:, 0:1]
        b = lg[:, 1:2]
        mx = jnp.maximum(a, b)
        lse = jnp.log(jnp.exp(a - mx) + jnp.exp(b - mx))
        sel = b if want_pos else a
        return (sel - mx) - lse

    partial = (jnp.sum(logp(lg0, True)) + jnp.sum(logp(lg1, False))
               + jnp.sum(logp(lg2, False)))

    @pl.when(i == 0)
    def _():
        loss_ref[...] = jnp.zeros_like(loss_ref)

    loss_ref[...] += jnp.full((1, 1), partial, jnp.float32)

    @pl.when(i == NBLK - 1)
    def _():
        loss_ref[...] = loss_ref[...] * (-1.0 / (3.0 * B))


@functools.partial(jax.jit, static_argnames=())
def kernel(all_image_features, all_text_features, logits_per_image,
           logits_per_text, proj_w, proj_b):
    pw_img = jnp.zeros((D, PAD), jnp.float32).at[:, :2].set(proj_w[:D])
    pw_txt = jnp.zeros((D, PAD), jnp.float32).at[:, :2].set(proj_w[D:])
    pb_pad = jnp.zeros((1, PAD), jnp.float32).at[0, :2].set(proj_b)

    row_spec = pl.BlockSpec((R, B), lambda i: (i, 0))
    feat_spec = pl.BlockSpec((R, D), lambda i: (i, 0))
    full_w = pl.BlockSpec((D, PAD), lambda i: (0, 0))
    idx_spec = pl.BlockSpec((1, B), lambda i: (0, 0))
    proj_out = pl.BlockSpec((R, PAD), lambda i: (i, 0))

    idxt, idxi, pi, pt = pl.pallas_call(
        _sample_project_body,
        grid=(NBLK,),
        in_specs=[row_spec, row_spec,
                  feat_spec, feat_spec, full_w, full_w],
        out_specs=[idx_spec, idx_spec, proj_out, proj_out],
        out_shape=[
            jax.ShapeDtypeStruct((1, B), jnp.int32),
            jax.ShapeDtypeStruct((1, B), jnp.int32),
            jax.ShapeDtypeStruct((B, PAD), jnp.float32),
            jax.ShapeDtypeStruct((B, PAD), jnp.float32),
        ],
        scratch_shapes=[pltpu.VMEM((R, B), jnp.uint32),
                        pltpu.VMEM((R, B), jnp.uint32)],
    )(logits_per_image, logits_per_text,
      all_image_features, all_text_features, pw_img, pw_txt)

    sc_gather = functools.partial(
        pl.kernel,
        mesh=plsc.VectorSubcoreMesh(core_axis_name="c", subcore_axis_name="s"),
        out_type=[
            jax.ShapeDtypeStruct((B, PAD), jnp.float32),
            jax.ShapeDtypeStruct((B, PAD), jnp.float32),
        ],
        scratch_types=[
            pltpu.VMEM((_BPW,), jnp.int32),
            pltpu.VMEM((_BPW, PAD), jnp.float32),
            pltpu.SemaphoreType.DMA,
        ],
    )(_sc_gather_body)
    gath_t, gath_i = sc_gather(pt, pi, idxt.reshape(B), idxi.reshape(B))

    pb_spec = pl.BlockSpec((1, PAD), lambda i: (0, 0))
    lg_spec = pl.BlockSpec((R, PAD), lambda i: (i, 0))
    loss_spec = pl.BlockSpec((1, 1), lambda i: (0, 0))

    lg0, lg1, lg2, loss = pl.pallas_call(
        _finalize_body,
        grid=(NBLK,),
        in_specs=[lg_spec, lg_spec, lg_spec, lg_spec, pb_spec],
        out_specs=[lg_spec, lg_spec, lg_spec, loss_spec],
        out_shape=[
            jax.ShapeDtypeStruct((B, PAD), jnp.float32),
            jax.ShapeDtypeStruct((B, PAD), jnp.float32),
            jax.ShapeDtypeStruct((B, PAD), jnp.float32),
            jax.ShapeDtypeStruct((1, 1), jnp.float32),
        ],
    )(pi, pt, gath_t, gath_i, pb_pad)

    logits = jnp.concatenate([lg0[:, :2], lg1[:, :2], lg2[:, :2]], axis=0)
    itm_labels = jnp.concatenate([
        jnp.ones((B,), dtype=jnp.int32),
        jnp.zeros((B,), dtype=jnp.int32),
        jnp.zeros((B,), dtype=jnp.int32),
    ])
    return loss[0, 0], logits, itm_labels


# speedup vs baseline: 1.1673x; 1.0042x over previous
import functools

import jax
import jax.numpy as jnp
from jax.experimental import pallas as pl
from jax.experimental.pallas import tpu as pltpu
from jax.experimental.pallas import tpu_sc as plsc

B = 4096
D = 512
R = 256
NBLK = B // R
PAD = 128

_K1 = (1832780943, 270669613)
_K2 = (64467757, 2916123636)
_TINY = float(jnp.finfo(jnp.float32).tiny)
_ROTS = ((13, 15, 26, 6), (17, 29, 16, 24))


def _gumbel_from_bits(bits):
    fb = jax.lax.shift_right_logical(bits, jnp.uint32(9)) | jnp.uint32(0x3F800000)
    f = jax.lax.bitcast_convert_type(fb, jnp.float32) - 1.0
    u = jnp.maximum(jnp.float32(_TINY), f * 1.0 + jnp.float32(_TINY))
    return -jnp.log(-jnp.log(u))


_CH = 64
_CW = 4096


def _threefry_chunks(r0, bs1_ref, bs2_ref):
    def sched(key):
        k0 = jnp.uint32(key[0])
        k1 = jnp.uint32(key[1])
        return (k0, k1, k0 ^ k1 ^ jnp.uint32(0x1BD11BDA))

    ksa, ksb = sched(_K1), sched(_K2)
    col_c = jax.lax.broadcasted_iota(jnp.int32, (_CH, _CW), 1)
    row_c = jax.lax.broadcasted_iota(jnp.int32, (_CH, _CW), 0)
    ncw = B // _CW

    def body(jk, carry):
        j = jk // ncw
        k = jk % ncw
        n = ((r0 + j * _CH + row_c) * B + (k * _CW + col_c)).astype(jnp.uint32)
        xa0 = jnp.full((_CH, _CW), ksa[0], jnp.uint32)
        xb0 = jnp.full((_CH, _CW), ksb[0], jnp.uint32)
        xa1 = n + ksa[1]
        xb1 = n + ksb[1]
        for i in range(5):
            for r in _ROTS[i % 2]:
                xa0 = xa0 + xa1
                xb0 = xb0 + xb1
                xa1 = (jax.lax.shift_left(xa1, jnp.uint32(r))
                       | jax.lax.shift_right_logical(xa1, jnp.uint32(32 - r))) ^ xa0
                xb1 = (jax.lax.shift_left(xb1, jnp.uint32(r))
                       | jax.lax.shift_right_logical(xb1, jnp.uint32(32 - r))) ^ xb0
            xa0 = xa0 + ksa[(i + 1) % 3]
            xb0 = xb0 + ksb[(i + 1) % 3]
            xa1 = xa1 + ksa[(i + 2) % 3] + jnp.uint32(i + 1)
            xb1 = xb1 + ksb[(i + 2) % 3] + jnp.uint32(i + 1)
        bs1_ref[pl.ds(j * _CH, _CH), pl.ds(k * _CW, _CW)] = xa0 ^ xa1
        bs2_ref[pl.ds(j * _CH, _CH), pl.ds(k * _CW, _CW)] = xb0 ^ xb1
        return carry

    jax.lax.fori_loop(0, (R // _CH) * ncw, body, 0)


def _sample_project_body(li_ref, lt_ref, ai_ref, at_ref,
                         pwi_ref, pwt_ref,
                         idxt_ref, idxi_ref, pi_ref, pt_ref,
                         bs1_ref, bs2_ref):
    i = pl.program_id(0)
    r0 = i * R

    _threefry_chunks(r0, bs1_ref, bs2_ref)

    col = jax.lax.broadcasted_iota(jnp.int32, (R, B), 1)
    row = r0 + jax.lax.broadcasted_iota(jnp.int32, (R, B), 0)
    diag = col == row

    def draw(x, bits):
        g = _gumbel_from_bits(bits)
        m = jnp.max(x, axis=1, keepdims=True)
        u = jnp.exp(x - m)
        s = jnp.sum(u, axis=1, keepdims=True)
        w = u / s
        w = jnp.where(diag, 0.0, w)
        v = jnp.where(w > 0, jnp.log(w), -jnp.inf) + g
        vmax = jnp.max(v, axis=1, keepdims=True)
        cand = jnp.where(v == vmax, col, B)
        return jnp.min(cand, axis=1).astype(jnp.int32)

    idxt_ref[0, pl.ds(r0, R)] = draw(li_ref[...], bs1_ref[...])
    idxi_ref[0, pl.ds(r0, R)] = draw(lt_ref[...], bs2_ref[...])

    pi_ref[...] = jnp.dot(ai_ref[...], pwi_ref[...],
                          preferred_element_type=jnp.float32)
    pt_ref[...] = jnp.dot(at_ref[...], pwt_ref[...],
                          preferred_element_type=jnp.float32)


_NW = 32
_BPW = B // _NW


def _sc_gather_body(pt_hbm, pi_hbm, idxt_hbm, idxi_hbm,
                    outt_hbm, outi_hbm, idx_v, rows_v, sem):
    wid = jax.lax.axis_index("s") * 2 + jax.lax.axis_index("c")
    base = wid * _BPW
    pltpu.sync_copy(idxt_hbm.at[pl.ds(base, _BPW)], idx_v)
    pltpu.async_copy(pt_hbm.at[idx_v], rows_v, sem).wait()
    pltpu.sync_copy(rows_v, outt_hbm.at[pl.ds(base, _BPW)])
    pltpu.sync_copy(idxi_hbm.at[pl.ds(base, _BPW)], idx_v)
    pltpu.async_copy(pi_hbm.at[idx_v], rows_v, sem).wait()
    pltpu.sync_copy(rows_v, outi_hbm.at[pl.ds(base, _BPW)])


def _finalize_body(pi_ref, pt_ref, gt_ref, gi_ref, pb_ref,
                   lg0_ref, lg1_ref, lg2_ref, loss_ref):
    i = pl.program_id(0)

    gath_t = gt_ref[...]
    gath_i = gi_ref[...]
    pi_blk = pi_ref[...]
    pt_blk = pt_ref[...]
    pb = pb_ref[...]

    lg0 = pi_blk + pt_blk + pb
    lg1 = pi_blk + gath_t + pb
    lg2 = gath_i + pt_blk + pb
    lg0_ref[...] = lg0
    lg1_ref[...] = lg1
    lg2_ref[...] = lg2

    def logp(lg, want_pos):
        a = lg[:, 0:1]
        b = lg[:, 1:2]
        mx = jnp.maximum(a, b)
        lse = jnp.log(jnp.exp(a - mx) + jnp.exp(b - mx))
        sel = b if want_pos else a
        return (sel - mx) - lse

    partial = (jnp.sum(logp(lg0, True)) + jnp.sum(logp(lg1, False))
               + jnp.sum(logp(lg2, False)))

    @pl.when(i == 0)
    def _():
        loss_ref[...] = jnp.zeros_like(loss_ref)

    loss_ref[...] += jnp.full((1, 1), partial, jnp.float32)

    @pl.when(i == NBLK - 1)
    def _():
        loss_ref[...] = loss_ref[...] * (-1.0 / (3.0 * B))


@functools.partial(jax.jit, static_argnames=())
def kernel(all_image_features, all_text_features, logits_per_image,
           logits_per_text, proj_w, proj_b):
    pw_img = jnp.zeros((D, PAD), jnp.float32).at[:, :2].set(proj_w[:D])
    pw_txt = jnp.zeros((D, PAD), jnp.float32).at[:, :2].set(proj_w[D:])
    pb_pad = jnp.zeros((1, PAD), jnp.float32).at[0, :2].set(proj_b)

    row_spec = pl.BlockSpec((R, B), lambda i: (i, 0))
    feat_spec = pl.BlockSpec((R, D), lambda i: (i, 0))
    full_w = pl.BlockSpec((D, PAD), lambda i: (0, 0))
    idx_spec = pl.BlockSpec((1, B), lambda i: (0, 0))
    proj_out = pl.BlockSpec((R, PAD), lambda i: (i, 0))

    idxt, idxi, pi, pt = pl.pallas_call(
        _sample_project_body,
        grid=(NBLK,),
        in_specs=[row_spec, row_spec,
                  feat_spec, feat_spec, full_w, full_w],
        out_specs=[idx_spec, idx_spec, proj_out, proj_out],
        out_shape=[
            jax.ShapeDtypeStruct((1, B), jnp.int32),
            jax.ShapeDtypeStruct((1, B), jnp.int32),
            jax.ShapeDtypeStruct((B, PAD), jnp.float32),
            jax.ShapeDtypeStruct((B, PAD), jnp.float32),
        ],
        scratch_shapes=[pltpu.VMEM((R, B), jnp.uint32),
                        pltpu.VMEM((R, B), jnp.uint32)],
    )(logits_per_image, logits_per_text,
      all_image_features, all_text_features, pw_img, pw_txt)

    sc_gather = functools.partial(
        pl.kernel,
        mesh=plsc.VectorSubcoreMesh(core_axis_name="c", subcore_axis_name="s"),
        out_type=[
            jax.ShapeDtypeStruct((B, PAD), jnp.float32),
            jax.ShapeDtypeStruct((B, PAD), jnp.float32),
        ],
        scratch_types=[
            pltpu.VMEM((_BPW,), jnp.int32),
            pltpu.VMEM((_BPW, PAD), jnp.float32),
            pltpu.SemaphoreType.DMA,
        ],
    )(_sc_gather_body)
    gath_t, gath_i = sc_gather(pt, pi, idxt.reshape(B), idxi.reshape(B))

    pb_spec = pl.BlockSpec((1, PAD), lambda i: (0, 0))
    lg_spec = pl.BlockSpec((R, PAD), lambda i: (i, 0))
    loss_spec = pl.BlockSpec((1, 1), lambda i: (0, 0))

    lg0, lg1, lg2, loss = pl.pallas_call(
        _finalize_body,
        grid=(NBLK,),
        in_specs=[lg_spec, lg_spec, lg_spec, lg_spec, pb_spec],
        out_specs=[lg_spec, lg_spec, lg_spec, loss_spec],
        out_shape=[
            jax.ShapeDtypeStruct((B, PAD), jnp.float32),
            jax.ShapeDtypeStruct((B, PAD), jnp.float32),
            jax.ShapeDtypeStruct((B, PAD), jnp.float32),
            jax.ShapeDtypeStruct((1, 1), jnp.float32),
        ],
    )(pi, pt, gath_t, gath_i, pb_pad)

    logits = jnp.concatenate([lg0[:, :2], lg1[:, :2], lg2[:, :2]], axis=0)
    itm_labels = jnp.concatenate([
        jnp.ones((B,), dtype=jnp.int32),
        jnp.zeros((B,), dtype=jnp.int32),
        jnp.zeros((B,), dtype=jnp.int32),
    ])
    return loss[0, 0], logits, itm_labels


# speedup vs baseline: 1.1702x; 1.0025x over previous
import functools

import jax
import jax.numpy as jnp
from jax.experimental import pallas as pl
from jax.experimental.pallas import tpu as pltpu
from jax.experimental.pallas import tpu_sc as plsc

B = 4096
D = 512
R = 256
NBLK = B // R
PAD = 128

_K1 = (1832780943, 270669613)
_K2 = (64467757, 2916123636)
_TINY = float(jnp.finfo(jnp.float32).tiny)
_ROTS = ((13, 15, 26, 6), (17, 29, 16, 24))


def _gumbel_from_bits(bits):
    fb = jax.lax.shift_right_logical(bits, jnp.uint32(9)) | jnp.uint32(0x3F800000)
    f = jax.lax.bitcast_convert_type(fb, jnp.float32) - 1.0
    u = jnp.maximum(jnp.float32(_TINY), f * 1.0 + jnp.float32(_TINY))
    return -jnp.log(-jnp.log(u))


_CH = 128
_CW = 4096


def _threefry_chunks(r0, bs1_ref, bs2_ref):
    def sched(key):
        k0 = jnp.uint32(key[0])
        k1 = jnp.uint32(key[1])
        return (k0, k1, k0 ^ k1 ^ jnp.uint32(0x1BD11BDA))

    ksa, ksb = sched(_K1), sched(_K2)
    col_c = jax.lax.broadcasted_iota(jnp.int32, (_CH, _CW), 1)
    row_c = jax.lax.broadcasted_iota(jnp.int32, (_CH, _CW), 0)
    ncw = B // _CW

    def body(jk, carry):
        j = jk // ncw
        k = jk % ncw
        n = ((r0 + j * _CH + row_c) * B + (k * _CW + col_c)).astype(jnp.uint32)
        xa0 = jnp.full((_CH, _CW), ksa[0], jnp.uint32)
        xb0 = jnp.full((_CH, _CW), ksb[0], jnp.uint32)
        xa1 = n + ksa[1]
        xb1 = n + ksb[1]
        for i in range(5):
            for r in _ROTS[i % 2]:
                xa0 = xa0 + xa1
                xb0 = xb0 + xb1
                xa1 = (jax.lax.shift_left(xa1, jnp.uint32(r))
                       | jax.lax.shift_right_logical(xa1, jnp.uint32(32 - r))) ^ xa0
                xb1 = (jax.lax.shift_left(xb1, jnp.uint32(r))
                       | jax.lax.shift_right_logical(xb1, jnp.uint32(32 - r))) ^ xb0
            xa0 = xa0 + ksa[(i + 1) % 3]
            xb0 = xb0 + ksb[(i + 1) % 3]
            xa1 = xa1 + ksa[(i + 2) % 3] + jnp.uint32(i + 1)
            xb1 = xb1 + ksb[(i + 2) % 3] + jnp.uint32(i + 1)
        bs1_ref[pl.ds(j * _CH, _CH), pl.ds(k * _CW, _CW)] = xa0 ^ xa1
        bs2_ref[pl.ds(j * _CH, _CH), pl.ds(k * _CW, _CW)] = xb0 ^ xb1
        return carry

    jax.lax.fori_loop(0, (R // _CH) * ncw, body, 0)


def _sample_project_body(li_ref, lt_ref, ai_ref, at_ref,
                         pwi_ref, pwt_ref,
                         idxt_ref, idxi_ref, pi_ref, pt_ref,
                         bs1_ref, bs2_ref):
    i = pl.program_id(0)
    r0 = i * R

    _threefry_chunks(r0, bs1_ref, bs2_ref)

    col = jax.lax.broadcasted_iota(jnp.int32, (R, B), 1)
    row = r0 + jax.lax.broadcasted_iota(jnp.int32, (R, B), 0)
    diag = col == row

    def draw(x, bits):
        g = _gumbel_from_bits(bits)
        m = jnp.max(x, axis=1, keepdims=True)
        u = jnp.exp(x - m)
        s = jnp.sum(u, axis=1, keepdims=True)
        w = u / s
        w = jnp.where(diag, 0.0, w)
        v = jnp.where(w > 0, jnp.log(w), -jnp.inf) + g
        vmax = jnp.max(v, axis=1, keepdims=True)
        cand = jnp.where(v == vmax, col, B)
        return jnp.min(cand, axis=1).astype(jnp.int32)

    idxt_ref[0, pl.ds(r0, R)] = draw(li_ref[...], bs1_ref[...])
    idxi_ref[0, pl.ds(r0, R)] = draw(lt_ref[...], bs2_ref[...])

    pi_ref[...] = jnp.dot(ai_ref[...], pwi_ref[...],
                          preferred_element_type=jnp.float32)
    pt_ref[...] = jnp.dot(at_ref[...], pwt_ref[...],
                          preferred_element_type=jnp.float32)


_NW = 32
_BPW = B // _NW


def _sc_gather_body(pt_hbm, pi_hbm, idxt_hbm, idxi_hbm,
                    outt_hbm, outi_hbm, idx_v, rows_v, sem):
    wid = jax.lax.axis_index("s") * 2 + jax.lax.axis_index("c")
    base = wid * _BPW
    pltpu.sync_copy(idxt_hbm.at[pl.ds(base, _BPW)], idx_v)
    pltpu.async_copy(pt_hbm.at[idx_v], rows_v, sem).wait()
    pltpu.sync_copy(rows_v, outt_hbm.at[pl.ds(base, _BPW)])
    pltpu.sync_copy(idxi_hbm.at[pl.ds(base, _BPW)], idx_v)
    pltpu.async_copy(pi_hbm.at[idx_v], rows_v, sem).wait()
    pltpu.sync_copy(rows_v, outi_hbm.at[pl.ds(base, _BPW)])


def _finalize_body(pi_ref, pt_ref, gt_ref, gi_ref, pb_ref,
                   lg0_ref, lg1_ref, lg2_ref, loss_ref):
    i = pl.program_id(0)

    gath_t = gt_ref[...]
    gath_i = gi_ref[...]
    pi_blk = pi_ref[...]
    pt_blk = pt_ref[...]
    pb = pb_ref[...]

    lg0 = pi_blk + pt_blk + pb
    lg1 = pi_blk + gath_t + pb
    lg2 = gath_i + pt_blk + pb
    lg0_ref[...] = lg0
    lg1_ref[...] = lg1
    lg2_ref[...] = lg2

    def logp(lg, want_pos):
        a = lg[:, 0:1]
        b = lg[:, 1:2]
        mx = jnp.maximum(a, b)
        lse = jnp.log(jnp.exp(a - mx) + jnp.exp(b - mx))
        sel = b if want_pos else a
        return (sel - mx) - lse

    partial = (jnp.sum(logp(lg0, True)) + jnp.sum(logp(lg1, False))
               + jnp.sum(logp(lg2, False)))

    @pl.when(i == 0)
    def _():
        loss_ref[...] = jnp.zeros_like(loss_ref)

    loss_ref[...] += jnp.full((1, 1), partial, jnp.float32)

    @pl.when(i == NBLK - 1)
    def _():
        loss_ref[...] = loss_ref[...] * (-1.0 / (3.0 * B))


@functools.partial(jax.jit, static_argnames=())
def kernel(all_image_features, all_text_features, logits_per_image,
           logits_per_text, proj_w, proj_b):
    pw_img = jnp.zeros((D, PAD), jnp.float32).at[:, :2].set(proj_w[:D])
    pw_txt = jnp.zeros((D, PAD), jnp.float32).at[:, :2].set(proj_w[D:])
    pb_pad = jnp.zeros((1, PAD), jnp.float32).at[0, :2].set(proj_b)

    row_spec = pl.BlockSpec((R, B), lambda i: (i, 0))
    feat_spec = pl.BlockSpec((R, D), lambda i: (i, 0))
    full_w = pl.BlockSpec((D, PAD), lambda i: (0, 0))
    idx_spec = pl.BlockSpec((1, B), lambda i: (0, 0))
    proj_out = pl.BlockSpec((R, PAD), lambda i: (i, 0))

    idxt, idxi, pi, pt = pl.pallas_call(
        _sample_project_body,
        grid=(NBLK,),
        in_specs=[row_spec, row_spec,
                  feat_spec, feat_spec, full_w, full_w],
        out_specs=[idx_spec, idx_spec, proj_out, proj_out],
        out_shape=[
            jax.ShapeDtypeStruct((1, B), jnp.int32),
            jax.ShapeDtypeStruct((1, B), jnp.int32),
            jax.ShapeDtypeStruct((B, PAD), jnp.float32),
            jax.ShapeDtypeStruct((B, PAD), jnp.float32),
        ],
        scratch_shapes=[pltpu.VMEM((R, B), jnp.uint32),
                        pltpu.VMEM((R, B), jnp.uint32)],
    )(logits_per_image, logits_per_text,
      all_image_features, all_text_features, pw_img, pw_txt)

    sc_gather = functools.partial(
        pl.kernel,
        mesh=plsc.VectorSubcoreMesh(core_axis_name="c", subcore_axis_name="s"),
        out_type=[
            jax.ShapeDtypeStruct((B, PAD), jnp.float32),
            jax.ShapeDtypeStruct((B, PAD), jnp.float32),
        ],
        scratch_types=[
            pltpu.VMEM((_BPW,), jnp.int32),
            pltpu.VMEM((_BPW, PAD), jnp.float32),
            pltpu.SemaphoreType.DMA,
        ],
    )(_sc_gather_body)
    gath_t, gath_i = sc_gather(pt, pi, idxt.reshape(B), idxi.reshape(B))

    pb_spec = pl.BlockSpec((1, PAD), lambda i: (0, 0))
    lg_spec = pl.BlockSpec((R, PAD), lambda i: (i, 0))
    loss_spec = pl.BlockSpec((1, 1), lambda i: (0, 0))

    lg0, lg1, lg2, loss = pl.pallas_call(
        _finalize_body,
        grid=(NBLK,),
        in_specs=[lg_spec, lg_spec, lg_spec, lg_spec, pb_spec],
        out_specs=[lg_spec, lg_spec, lg_spec, loss_spec],
        out_shape=[
            jax.ShapeDtypeStruct((B, PAD), jnp.float32),
            jax.ShapeDtypeStruct((B, PAD), jnp.float32),
            jax.ShapeDtypeStruct((B, PAD), jnp.float32),
            jax.ShapeDtypeStruct((1, 1), jnp.float32),
        ],
    )(pi, pt, gath_t, gath_i, pb_pad)

    logits = jnp.concatenate([lg0[:, :2], lg1[:, :2], lg2[:, :2]], axis=0)
    itm_labels = jnp.concatenate([
        jnp.ones((B,), dtype=jnp.int32),
        jnp.zeros((B,), dtype=jnp.int32),
        jnp.zeros((B,), dtype=jnp.int32),
    ])
    return loss[0, 0], logits, itm_labels
